# baseline (speedup 1.0000x reference)
import functools

import jax
import jax.numpy as jnp
from jax import lax
from jax.experimental import pallas as pl
from jax.experimental.pallas import tpu as pltpu
from jax.experimental.pallas import tpu_sc as plsc

_INPUT_DIM = 1024
_K_DIM = 512
_HALF = _K_DIM // 2
_HEADS = 4
_KNN = 16
_N_KEYS = 256
_V_DIM = 1024
_R = _HEADS * _KNN
_TOKENS = 2048

_TB = 256


_PAIRS = sorted((i * _KNN + j, i, j)
                for i in range(_KNN) for j in range(_KNN)
                if (i + 1) * (j + 1) <= _KNN)
_NCAND = 64
_PAIR_I = [p[1] for p in _PAIRS] + [0] * (_NCAND - len(_PAIRS))
_PAIR_J = [p[2] for p in _PAIRS] + [0] * (_NCAND - len(_PAIRS))
_PAIR_VALID = [1.0] * len(_PAIRS) + [0.0] * (_NCAND - len(_PAIRS))

import numpy as _np

_E1_NP = _np.zeros((_KNN, _NCAND), _np.float32)
_E2_NP = _np.zeros((_KNN, _NCAND), _np.float32)
for _l in range(len(_PAIRS)):
    _E1_NP[_PAIR_I[_l], _l] = 1.0
    _E2_NP[_PAIR_J[_l], _l] = 1.0
_PAD_NP = ((_np.asarray(_PAIR_VALID, _np.float32) - 1.0)
           * _np.float32(1e30)).reshape(1, _NCAND)


def _topk16_exact(s, payload=None):
    T, L = s.shape
    lane = lax.broadcasted_iota(jnp.int32, (T, L), 1)
    col = lax.broadcasted_iota(jnp.int32, (T, _KNN), 1)
    vals = jnp.zeros((T, _KNN), jnp.float32)
    idxs = jnp.zeros((T, _KNN), jnp.int32)
    pays = jnp.zeros((T, _KNN), jnp.float32) if payload is not None else None
    neg = jnp.float32(-jnp.inf)
    big = jnp.int32(2 ** 30)
    for k in range(_KNN):
        m = jnp.max(s, axis=1, keepdims=True)
        mask = s == m
        isel = jnp.min(jnp.where(mask, lane, big), axis=1, keepdims=True)
        onehot = lane == isel
        s = jnp.where(onehot, neg, s)
        vals = jnp.where(col == k, m, vals)
        idxs = jnp.where(col == k, isel, idxs)
        if payload is not None:
            psel = jnp.sum(jnp.where(onehot, payload, 0.0), axis=1,
                           keepdims=True)
            pays = jnp.where(col == k, psel, pays)
    return vals, idxs, pays


def _score_body(x_ref, keys_ref, Wq_ref, bq_ref,
                E1_ref, E2_ref, pad_ref, idx_ref, w_ref):
    x = x_ref[...]
    dn = (((1,), (1,)), ((), ()))
    hi = lax.Precision.HIGHEST
    bf = jnp.bfloat16
    q = lax.dot_general(x.astype(bf), Wq_ref[...].astype(bf), dn,
                        preferred_element_type=jnp.float32) + bq_ref[...]

    keys = keys_ref[...]
    s1_list, s2_list = [], []
    for h in range(_HEADS):
        q1 = q[:, h * _K_DIM: h * _K_DIM + _HALF].astype(bf)
        q2 = q[:, h * _K_DIM + _HALF: (h + 1) * _K_DIM].astype(bf)
        K1 = keys[(2 * h) * _N_KEYS: (2 * h + 1) * _N_KEYS, :].astype(bf)
        K2 = keys[(2 * h + 1) * _N_KEYS: (2 * h + 2) * _N_KEYS, :].astype(bf)
        s1_list.append(lax.dot_general(q1, K1, dn,
                                       preferred_element_type=jnp.float32))
        s2_list.append(lax.dot_general(q2, K2, dn,
                                       preferred_element_type=jnp.float32))
    s12 = jnp.concatenate(s1_list + s2_list, axis=0)
    v12, i12, _ = _topk16_exact(s12)
    half_rows = _HEADS * _TB
    v1 = v12[:half_rows, :]
    v2 = v12[half_rows:, :]
    i1 = i12[:half_rows, :]
    i2 = i12[half_rows:, :]

    E1 = E1_ref[...]
    E2 = E2_ref[...]
    pad = pad_ref[...]
    dc = (((1,), (0,)), ((), ()))
    c = (lax.dot_general(v1, E1, dc, precision=hi)
         + lax.dot_general(v2, E2, dc, precision=hi) + pad)
    cidx = (lax.dot_general(i1.astype(jnp.float32) * float(_N_KEYS), E1, dc,
                            precision=hi)
            + lax.dot_general(i2.astype(jnp.float32), E2, dc, precision=hi))
    cv, _, csel = _topk16_exact(c, cidx)

    e = jnp.exp(cv - cv[:, 0:1])
    w = e / jnp.sum(e, axis=1, keepdims=True)

    idx_all = csel.astype(jnp.int32)
    idx_ref[...] = jnp.concatenate(
        [idx_all[h * _TB:(h + 1) * _TB, :] for h in range(_HEADS)], axis=1)
    w_ref[...] = jnp.concatenate(
        [w[h * _TB:(h + 1) * _TB, :] for h in range(_HEADS)], axis=1)


def _score_call(x, keys, Wq, bq2):
    n = x.shape[0] // _TB
    return pl.pallas_call(
        _score_body,
        grid=(n,),
        in_specs=[
            pl.BlockSpec((_TB, _INPUT_DIM), lambda i: (i, 0)),
            pl.BlockSpec((2 * _HEADS * _N_KEYS, _HALF), lambda i: (0, 0)),
            pl.BlockSpec((_HEADS * _K_DIM, _INPUT_DIM), lambda i: (0, 0)),
            pl.BlockSpec((1, _HEADS * _K_DIM), lambda i: (0, 0)),
            pl.BlockSpec((_KNN, _NCAND), lambda i: (0, 0)),
            pl.BlockSpec((_KNN, _NCAND), lambda i: (0, 0)),
            pl.BlockSpec((1, _NCAND), lambda i: (0, 0)),
        ],
        out_specs=[
            pl.BlockSpec((_TB, _R), lambda i: (i, 0)),
            pl.BlockSpec((_TB, _R), lambda i: (i, 0)),
        ],
        out_shape=[
            jax.ShapeDtypeStruct((x.shape[0], _R), jnp.int32),
            jax.ShapeDtypeStruct((x.shape[0], _R), jnp.float32),
        ],
    )(x, keys, Wq, bq2,
      jnp.asarray(_E1_NP), jnp.asarray(_E2_NP), jnp.asarray(_PAD_NP))


def _proj_body(y_ref, x_ref, Wsw_ref, bsw_ref, Wvp_ref, bvp_ref, out_ref):
    dn = (((1,), (1,)), ((), ()))
    a = lax.dot_general(x_ref[...], Wsw_ref[...], dn) + bsw_ref[...]
    h = y_ref[...] * (a * jax.nn.sigmoid(a))
    out_ref[...] = lax.dot_general(h, Wvp_ref[...], dn) + bvp_ref[...]


def _proj_call(y, xc, Wsw, bsw2, Wvp, bvp2):
    n = y.shape[0] // _TB
    return pl.pallas_call(
        _proj_body,
        grid=(n,),
        in_specs=[
            pl.BlockSpec((_TB, _V_DIM), lambda i: (i, 0)),
            pl.BlockSpec((_TB, _INPUT_DIM), lambda i: (i, 0)),
            pl.BlockSpec((_V_DIM, _INPUT_DIM), lambda i: (0, 0)),
            pl.BlockSpec((1, _V_DIM), lambda i: (0, 0)),
            pl.BlockSpec((_V_DIM, _V_DIM), lambda i: (0, 0)),
            pl.BlockSpec((1, _V_DIM), lambda i: (0, 0)),
        ],
        out_specs=pl.BlockSpec((_TB, _V_DIM), lambda i: (i, 0)),
        out_shape=jax.ShapeDtypeStruct((y.shape[0], _V_DIM), jnp.float32),
    )(y, xc, Wsw, bsw2, Wvp, bvp2)


_NC = 2
_NS = 16
_NW = _NC * _NS
_CHUNKS = (256, 256, 512, 512, 512)


def _bcast16(vec, j):
    idx = jnp.full((16, 1), j, jnp.int32)
    return lax.gather(
        vec, idx,
        lax.GatherDimensionNumbers(offset_dims=(), collapsed_slice_dims=(0,),
                                   start_index_map=(0,)),
        (1,), mode=lax.GatherScatterMode.PROMISE_IN_BOUNDS)


_RH = _R // 2


def _make_bag_body(tpw):
  def _bag_body(values_hbm, idx_hbm, w_hbm, out_hbm, idx_v, w_v,
                buf0, buf1, acc_a, acc_b, sem0, sem1, osem_a, osem_b):
    wid = lax.axis_index("s") * _NC + lax.axis_index("c")
    base = wid * tpw
    pltpu.sync_copy(idx_hbm.at[pl.ds(base, tpw)], idx_v)
    pltpu.sync_copy(w_hbm.at[pl.ds(base, tpw)], w_v)

    def half_src(t, h):
        return values_hbm.at[idx_v.at[t, pl.ds(h * _RH, _RH)]]

    def accum_half(buf, t, h, first, acc_v):
        wc = [w_v[t, pl.ds(h * _RH + g * 16, 16)] for g in range(_RH // 16)]

        def chunk_body(cc, inner):
            sl = pl.ds(cc * 16, 16)
            parts = [jnp.zeros((16,), jnp.float32) for _ in range(4)]
            for g in range(_RH // 16):
                for j in range(16):
                    r = g * 16 + j
                    wb = _bcast16(wc[g], j)
                    parts[r % 4] = parts[r % 4] + wb * buf[r, sl]
            acc = (parts[0] + parts[1]) + (parts[2] + parts[3])
            if first:
                acc_v[sl] = acc
            else:
                acc_v[sl] += acc
            return inner

        lax.fori_loop(0, _V_DIM // 16, chunk_body, 0)

    pltpu.async_copy(half_src(0, 0), buf0, sem0)

    def one_token(t, acc, osem, wait_prev):
        pltpu.async_copy(half_src(t, 1), buf1, sem1)
        pltpu.make_async_copy(half_src(t, 0), buf0, sem0).wait()
        if wait_prev is not None:
            @pl.when(wait_prev)
            def _():
                pltpu.make_async_copy(acc, out_hbm.at[base + t - 2],
                                      osem).wait()
        accum_half(buf0, t, 0, True, acc)
        tn = jnp.minimum(t + 1, tpw - 1)
        pltpu.async_copy(half_src(tn, 0), buf0, sem0)
        pltpu.make_async_copy(half_src(t, 1), buf1, sem1).wait()
        accum_half(buf1, t, 1, False, acc)
        pltpu.async_copy(acc, out_hbm.at[base + t], osem)

    def pair_body(n, carry):
        one_token(2 * n, acc_a, osem_a, n > 0)
        one_token(2 * n + 1, acc_b, osem_b, n > 0)
        return carry

    lax.fori_loop(0, tpw // 2, pair_body, 0)
    pltpu.make_async_copy(half_src(tpw - 1, 0), buf0, sem0).wait()
    pltpu.make_async_copy(acc_a, out_hbm.at[base + tpw - 2], osem_a).wait()
    pltpu.make_async_copy(acc_b, out_hbm.at[base + tpw - 1], osem_b).wait()
  return _bag_body


@functools.cache
def _get_bag(chunk):
    tpw = chunk // _NW
    return pl.kernel(
        _make_bag_body(tpw),
        out_type=jax.ShapeDtypeStruct((chunk, _V_DIM), jnp.float32),
        mesh=plsc.VectorSubcoreMesh(core_axis_name="c", subcore_axis_name="s"),
        scratch_types=[
            pltpu.VMEM((tpw, _R), jnp.int32),
            pltpu.VMEM((tpw, _R), jnp.float32),
            pltpu.VMEM((_RH, _V_DIM), jnp.float32),
            pltpu.VMEM((_RH, _V_DIM), jnp.float32),
            pltpu.VMEM((_V_DIM,), jnp.float32),
            pltpu.VMEM((_V_DIM,), jnp.float32),
            pltpu.SemaphoreType.DMA,
            pltpu.SemaphoreType.DMA,
            pltpu.SemaphoreType.DMA,
            pltpu.SemaphoreType.DMA,
        ],
    )


def _bag(values, idx, w):
    return _get_bag(idx.shape[0])(values, idx, w)


def kernel(x, keys, values, Wq, bq, Wvp, bvp, Wsw, bsw):
    bq2 = bq.reshape(1, -1)
    bsw2 = bsw.reshape(1, -1)
    bvp2 = bvp.reshape(1, -1)
    outs = []
    off = 0
    for chunk in _CHUNKS:
        xc = lax.slice_in_dim(x, off, off + chunk, axis=0)
        idx, w = _score_call(xc, keys, Wq, bq2)
        y = _bag(values, idx, w)
        outs.append(_proj_call(y, xc, Wsw, bsw2, Wvp, bvp2))
        off += chunk
    return jnp.concatenate(outs, axis=0)


# speedup vs baseline: 1.0158x; 1.0158x over previous
import functools

import jax
import jax.numpy as jnp
from jax import lax
from jax.experimental import pallas as pl
from jax.experimental.pallas import tpu as pltpu
from jax.experimental.pallas import tpu_sc as plsc

_INPUT_DIM = 1024
_K_DIM = 512
_HALF = _K_DIM // 2
_HEADS = 4
_KNN = 16
_N_KEYS = 256
_V_DIM = 1024
_R = _HEADS * _KNN
_TOKENS = 2048

_TB = 256


_PAIRS = sorted((i * _KNN + j, i, j)
                for i in range(_KNN) for j in range(_KNN)
                if (i + 1) * (j + 1) <= _KNN)
_NCAND = 64
_PAIR_I = [p[1] for p in _PAIRS] + [0] * (_NCAND - len(_PAIRS))
_PAIR_J = [p[2] for p in _PAIRS] + [0] * (_NCAND - len(_PAIRS))
_PAIR_VALID = [1.0] * len(_PAIRS) + [0.0] * (_NCAND - len(_PAIRS))

import numpy as _np

_E1_NP = _np.zeros((_KNN, _NCAND), _np.float32)
_E2_NP = _np.zeros((_KNN, _NCAND), _np.float32)
for _l in range(len(_PAIRS)):
    _E1_NP[_PAIR_I[_l], _l] = 1.0
    _E2_NP[_PAIR_J[_l], _l] = 1.0
_PAD_NP = ((_np.asarray(_PAIR_VALID, _np.float32) - 1.0)
           * _np.float32(1e30)).reshape(1, _NCAND)


def _topk16_exact(s, payload=None):
    T, L = s.shape
    lane = lax.broadcasted_iota(jnp.int32, (T, L), 1)
    col = lax.broadcasted_iota(jnp.int32, (T, _KNN), 1)
    vals = jnp.zeros((T, _KNN), jnp.float32)
    idxs = jnp.zeros((T, _KNN), jnp.int32)
    pays = jnp.zeros((T, _KNN), jnp.float32) if payload is not None else None
    neg = jnp.float32(-jnp.inf)
    big = jnp.int32(2 ** 30)
    for k in range(_KNN):
        m = jnp.max(s, axis=1, keepdims=True)
        mask = s == m
        isel = jnp.min(jnp.where(mask, lane, big), axis=1, keepdims=True)
        onehot = lane == isel
        s = jnp.where(onehot, neg, s)
        vals = jnp.where(col == k, m, vals)
        idxs = jnp.where(col == k, isel, idxs)
        if payload is not None:
            psel = jnp.sum(jnp.where(onehot, payload, 0.0), axis=1,
                           keepdims=True)
            pays = jnp.where(col == k, psel, pays)
    return vals, idxs, pays


def _score_body(x_ref, keys_ref, Wq_ref, bq_ref,
                E1_ref, E2_ref, pad_ref, idx_ref, w_ref):
    x = x_ref[...]
    dn = (((1,), (1,)), ((), ()))
    hi = lax.Precision.HIGHEST
    bf = jnp.bfloat16
    q = lax.dot_general(x.astype(bf), Wq_ref[...].astype(bf), dn,
                        preferred_element_type=jnp.float32) + bq_ref[...]

    keys = keys_ref[...]
    s1_list, s2_list = [], []
    for h in range(_HEADS):
        q1 = q[:, h * _K_DIM: h * _K_DIM + _HALF].astype(bf)
        q2 = q[:, h * _K_DIM + _HALF: (h + 1) * _K_DIM].astype(bf)
        K1 = keys[(2 * h) * _N_KEYS: (2 * h + 1) * _N_KEYS, :].astype(bf)
        K2 = keys[(2 * h + 1) * _N_KEYS: (2 * h + 2) * _N_KEYS, :].astype(bf)
        s1_list.append(lax.dot_general(q1, K1, dn,
                                       preferred_element_type=jnp.float32))
        s2_list.append(lax.dot_general(q2, K2, dn,
                                       preferred_element_type=jnp.float32))
    s12 = jnp.concatenate(s1_list + s2_list, axis=0)
    v12, i12, _ = _topk16_exact(s12)
    half_rows = _HEADS * _TB
    v1 = v12[:half_rows, :]
    v2 = v12[half_rows:, :]
    i1 = i12[:half_rows, :]
    i2 = i12[half_rows:, :]

    E1 = E1_ref[...]
    E2 = E2_ref[...]
    pad = pad_ref[...]
    dc = (((1,), (0,)), ((), ()))
    c = (lax.dot_general(v1, E1, dc, precision=hi)
         + lax.dot_general(v2, E2, dc, precision=hi) + pad)
    cidx = (lax.dot_general(i1.astype(jnp.float32) * float(_N_KEYS), E1, dc,
                            precision=hi)
            + lax.dot_general(i2.astype(jnp.float32), E2, dc, precision=hi))
    cv, _, csel = _topk16_exact(c, cidx)

    e = jnp.exp(cv - cv[:, 0:1])
    w = e / jnp.sum(e, axis=1, keepdims=True)

    idx_all = csel.astype(jnp.int32)
    idx_ref[...] = jnp.concatenate(
        [idx_all[h * _TB:(h + 1) * _TB, :] for h in range(_HEADS)], axis=1)
    w_ref[...] = jnp.concatenate(
        [w[h * _TB:(h + 1) * _TB, :] for h in range(_HEADS)], axis=1)


def _score_call(x, keys, Wq, bq2):
    n = x.shape[0] // _TB
    return pl.pallas_call(
        _score_body,
        grid=(n,),
        in_specs=[
            pl.BlockSpec((_TB, _INPUT_DIM), lambda i: (i, 0)),
            pl.BlockSpec((2 * _HEADS * _N_KEYS, _HALF), lambda i: (0, 0)),
            pl.BlockSpec((_HEADS * _K_DIM, _INPUT_DIM), lambda i: (0, 0)),
            pl.BlockSpec((1, _HEADS * _K_DIM), lambda i: (0, 0)),
            pl.BlockSpec((_KNN, _NCAND), lambda i: (0, 0)),
            pl.BlockSpec((_KNN, _NCAND), lambda i: (0, 0)),
            pl.BlockSpec((1, _NCAND), lambda i: (0, 0)),
        ],
        out_specs=[
            pl.BlockSpec((_TB, _R), lambda i: (i, 0)),
            pl.BlockSpec((_TB, _R), lambda i: (i, 0)),
        ],
        out_shape=[
            jax.ShapeDtypeStruct((x.shape[0], _R), jnp.int32),
            jax.ShapeDtypeStruct((x.shape[0], _R), jnp.float32),
        ],
    )(x, keys, Wq, bq2,
      jnp.asarray(_E1_NP), jnp.asarray(_E2_NP), jnp.asarray(_PAD_NP))


def _proj_body(y_ref, x_ref, Wsw_ref, bsw_ref, Wvp_ref, bvp_ref, out_ref):
    dn = (((1,), (1,)), ((), ()))
    a = lax.dot_general(x_ref[...], Wsw_ref[...], dn) + bsw_ref[...]
    h = y_ref[...] * (a * jax.nn.sigmoid(a))
    out_ref[...] = lax.dot_general(h, Wvp_ref[...], dn) + bvp_ref[...]


def _proj_call(y, xc, Wsw, bsw2, Wvp, bvp2):
    n = y.shape[0] // _TB
    return pl.pallas_call(
        _proj_body,
        grid=(n,),
        in_specs=[
            pl.BlockSpec((_TB, _V_DIM), lambda i: (i, 0)),
            pl.BlockSpec((_TB, _INPUT_DIM), lambda i: (i, 0)),
            pl.BlockSpec((_V_DIM, _INPUT_DIM), lambda i: (0, 0)),
            pl.BlockSpec((1, _V_DIM), lambda i: (0, 0)),
            pl.BlockSpec((_V_DIM, _V_DIM), lambda i: (0, 0)),
            pl.BlockSpec((1, _V_DIM), lambda i: (0, 0)),
        ],
        out_specs=pl.BlockSpec((_TB, _V_DIM), lambda i: (i, 0)),
        out_shape=jax.ShapeDtypeStruct((y.shape[0], _V_DIM), jnp.float32),
    )(y, xc, Wsw, bsw2, Wvp, bvp2)


_NC = 2
_NS = 16
_NW = _NC * _NS
_CHUNKS = (512, 512, 512, 512)


def _bcast16(vec, j):
    idx = jnp.full((16, 1), j, jnp.int32)
    return lax.gather(
        vec, idx,
        lax.GatherDimensionNumbers(offset_dims=(), collapsed_slice_dims=(0,),
                                   start_index_map=(0,)),
        (1,), mode=lax.GatherScatterMode.PROMISE_IN_BOUNDS)


_RH = _R // 2


def _make_bag_body(tpw):
  def _bag_body(values_hbm, idx_hbm, w_hbm, out_hbm, idx_v, w_v,
                buf0, buf1, acc_a, acc_b, sem0, sem1, osem_a, osem_b):
    wid = lax.axis_index("s") * _NC + lax.axis_index("c")
    base = wid * tpw
    pltpu.sync_copy(idx_hbm.at[pl.ds(base, tpw)], idx_v)
    pltpu.sync_copy(w_hbm.at[pl.ds(base, tpw)], w_v)

    def half_src(t, h):
        return values_hbm.at[idx_v.at[t, pl.ds(h * _RH, _RH)]]

    def accum_half(buf, t, h, first, acc_v):
        wc = [w_v[t, pl.ds(h * _RH + g * 16, 16)] for g in range(_RH // 16)]

        def chunk_body(cc, inner):
            sl = pl.ds(cc * 16, 16)
            parts = [jnp.zeros((16,), jnp.float32) for _ in range(4)]
            for g in range(_RH // 16):
                for j in range(16):
                    r = g * 16 + j
                    wb = _bcast16(wc[g], j)
                    parts[r % 4] = parts[r % 4] + wb * buf[r, sl]
            acc = (parts[0] + parts[1]) + (parts[2] + parts[3])
            if first:
                acc_v[sl] = acc
            else:
                acc_v[sl] += acc
            return inner

        lax.fori_loop(0, _V_DIM // 16, chunk_body, 0)

    pltpu.async_copy(half_src(0, 0), buf0, sem0)

    def one_token(t, acc, osem, wait_prev):
        pltpu.async_copy(half_src(t, 1), buf1, sem1)
        pltpu.make_async_copy(half_src(t, 0), buf0, sem0).wait()
        if wait_prev is not None:
            @pl.when(wait_prev)
            def _():
                pltpu.make_async_copy(acc, out_hbm.at[base + t - 2],
                                      osem).wait()
        accum_half(buf0, t, 0, True, acc)
        tn = jnp.minimum(t + 1, tpw - 1)
        pltpu.async_copy(half_src(tn, 0), buf0, sem0)
        pltpu.make_async_copy(half_src(t, 1), buf1, sem1).wait()
        accum_half(buf1, t, 1, False, acc)
        pltpu.async_copy(acc, out_hbm.at[base + t], osem)

    def pair_body(n, carry):
        one_token(2 * n, acc_a, osem_a, n > 0)
        one_token(2 * n + 1, acc_b, osem_b, n > 0)
        return carry

    lax.fori_loop(0, tpw // 2, pair_body, 0)
    pltpu.make_async_copy(half_src(tpw - 1, 0), buf0, sem0).wait()
    pltpu.make_async_copy(acc_a, out_hbm.at[base + tpw - 2], osem_a).wait()
    pltpu.make_async_copy(acc_b, out_hbm.at[base + tpw - 1], osem_b).wait()
  return _bag_body


@functools.cache
def _get_bag(chunk):
    tpw = chunk // _NW
    return pl.kernel(
        _make_bag_body(tpw),
        out_type=jax.ShapeDtypeStruct((chunk, _V_DIM), jnp.float32),
        mesh=plsc.VectorSubcoreMesh(core_axis_name="c", subcore_axis_name="s"),
        scratch_types=[
            pltpu.VMEM((tpw, _R), jnp.int32),
            pltpu.VMEM((tpw, _R), jnp.float32),
            pltpu.VMEM((_RH, _V_DIM), jnp.float32),
            pltpu.VMEM((_RH, _V_DIM), jnp.float32),
            pltpu.VMEM((_V_DIM,), jnp.float32),
            pltpu.VMEM((_V_DIM,), jnp.float32),
            pltpu.SemaphoreType.DMA,
            pltpu.SemaphoreType.DMA,
            pltpu.SemaphoreType.DMA,
            pltpu.SemaphoreType.DMA,
        ],
    )


def _bag(values, idx, w):
    return _get_bag(idx.shape[0])(values, idx, w)


def kernel(x, keys, values, Wq, bq, Wvp, bvp, Wsw, bsw):
    bq2 = bq.reshape(1, -1)
    bsw2 = bsw.reshape(1, -1)
    bvp2 = bvp.reshape(1, -1)
    xcs, scored = [], []
    off = 0
    for chunk in _CHUNKS:
        xc = lax.slice_in_dim(x, off, off + chunk, axis=0)
        xcs.append(xc)
        scored.append(_score_call(xc, keys, Wq, bq2))
        off += chunk
    ys = [_bag(values, idx, w) for idx, w in scored]
    outs = [_proj_call(y, xc, Wsw, bsw2, Wvp, bvp2)
            for y, xc in zip(ys, xcs)]
    return jnp.concatenate(outs, axis=0)
